# Initial kernel scaffold; baseline (speedup 1.0000x reference)
import functools

import jax
import jax.numpy as jnp
from jax import lax
from jax.experimental import pallas as pl
from jax.experimental.pallas import tpu as pltpu
from jax.experimental.pallas import tpu_sc as plsc

CHUNK = 1024


def kernel(indices, primitives):
    B, L = indices.shape
    V, D = primitives.shape
    N = B * L
    flat_idx = indices.reshape(N)

    info = plsc.get_sparse_core_info()
    NC, NS = info.num_cores, info.num_subcores
    NW = NC * NS
    n_per_w = N // NW
    n_chunks = n_per_w // CHUNK

    mesh = plsc.VectorSubcoreMesh(core_axis_name="c", subcore_axis_name="s")

    @functools.partial(
        pl.kernel,
        mesh=mesh,
        out_type=jax.ShapeDtypeStruct((N, D), jnp.float32),
        scratch_types=[
            pltpu.VMEM((CHUNK,), jnp.int32),
            pltpu.VMEM((CHUNK, D), jnp.float32),
            pltpu.SemaphoreType.DMA,
        ],
    )
    def gather_k(table_hbm, idx_hbm, out_hbm, idx_v, rows_v, sem):
        wid = lax.axis_index("s") * NC + lax.axis_index("c")
        base = wid * n_per_w

        def body(i, carry):
            off = base + i * CHUNK
            pltpu.sync_copy(idx_hbm.at[pl.ds(off, CHUNK)], idx_v)
            pltpu.async_copy(table_hbm.at[idx_v], rows_v, sem).wait()
            pltpu.sync_copy(rows_v, out_hbm.at[pl.ds(off, CHUNK)])
            return carry

        lax.fori_loop(0, n_chunks, body, 0)

    out = gather_k(primitives, flat_idx)
    return out.reshape(B, L, D)


# speedup vs baseline: 5.1476x; 5.1476x over previous
import functools

import jax
import jax.numpy as jnp
from jax import lax
from jax.experimental import pallas as pl
from jax.experimental.pallas import tpu as pltpu
from jax.experimental.pallas import tpu_sc as plsc

CHUNK = 1024


def kernel(indices, primitives):
    B, L = indices.shape
    V, D = primitives.shape
    N = B * L
    flat_idx = indices.reshape(N)

    info = plsc.get_sparse_core_info()
    NC, NS = info.num_cores, info.num_subcores
    NW = NC * NS
    n_per_w = N // NW
    n_chunks = n_per_w // CHUNK

    mesh = plsc.VectorSubcoreMesh(core_axis_name="c", subcore_axis_name="s")

    @functools.partial(
        pl.kernel,
        mesh=mesh,
        compiler_params=pltpu.CompilerParams(use_tc_tiling_on_sc=False),
        out_type=jax.ShapeDtypeStruct((N, D), jnp.float32),
        scratch_types=[
            pltpu.VMEM((CHUNK,), jnp.int32),
            pltpu.VMEM((CHUNK, D), jnp.float32),
            pltpu.SemaphoreType.DMA,
        ],
    )
    def gather_k(table_hbm, idx_hbm, out_hbm, idx_v, rows_v, sem):
        wid = lax.axis_index("s") * NC + lax.axis_index("c")
        base = wid * n_per_w

        def body(i, carry):
            off = base + i * CHUNK
            pltpu.sync_copy(idx_hbm.at[pl.ds(off, CHUNK)], idx_v)
            pltpu.async_copy(table_hbm.at[idx_v], rows_v, sem).wait()
            pltpu.sync_copy(rows_v, out_hbm.at[pl.ds(off, CHUNK)])
            return carry

        lax.fori_loop(0, n_chunks, body, 0)

    out = gather_k(primitives, flat_idx)
    return out.reshape(B, L, D)


# speedup vs baseline: 5.2721x; 1.0242x over previous
import functools

import jax
import jax.numpy as jnp
from jax import lax
from jax.experimental import pallas as pl
from jax.experimental.pallas import tpu as pltpu
from jax.experimental.pallas import tpu_sc as plsc

CHUNK = 512


def kernel(indices, primitives):
    B, L = indices.shape
    V, D = primitives.shape
    N = B * L
    flat_idx = indices.reshape(N)

    info = plsc.get_sparse_core_info()
    NC, NS = info.num_cores, info.num_subcores
    NW = NC * NS
    n_per_w = N // NW
    n_chunks = n_per_w // CHUNK

    mesh = plsc.VectorSubcoreMesh(core_axis_name="c", subcore_axis_name="s")

    @functools.partial(
        pl.kernel,
        mesh=mesh,
        compiler_params=pltpu.CompilerParams(use_tc_tiling_on_sc=False),
        out_type=jax.ShapeDtypeStruct((N, D), jnp.float32),
        scratch_types=[
            pltpu.VMEM((n_per_w,), jnp.int32),
            pltpu.VMEM((2, CHUNK, D), jnp.float32),
            pltpu.SemaphoreType.DMA((2,)),
            pltpu.SemaphoreType.DMA((2,)),
        ],
    )
    def gather_k(table_hbm, idx_hbm, out_hbm, idx_v, rows_v, gsem, ssem):
        wid = lax.axis_index("s") * NC + lax.axis_index("c")
        base = wid * n_per_w

        pltpu.sync_copy(idx_hbm.at[pl.ds(base, n_per_w)], idx_v)

        def fire_gather(i, s):
            pltpu.async_copy(
                table_hbm.at[idx_v.at[pl.ds(i * CHUNK, CHUNK)]],
                rows_v.at[s], gsem.at[s])

        def wait_gather(s):
            pltpu.make_async_copy(
                table_hbm.at[idx_v.at[pl.ds(0, CHUNK)]],
                rows_v.at[s], gsem.at[s]).wait()

        def fire_store(i, s):
            pltpu.async_copy(
                rows_v.at[s], out_hbm.at[pl.ds(base + i * CHUNK, CHUNK)],
                ssem.at[s])

        def wait_store(s):
            pltpu.make_async_copy(
                rows_v.at[s], out_hbm.at[pl.ds(base, CHUNK)],
                ssem.at[s]).wait()

        fire_gather(0, 0)
        wait_gather(0)
        fire_store(0, 0)
        fire_gather(1, 1)

        def body(g, carry):
            for b in range(2):
                i = 2 * g + 1 + b
                s = (1 + b) % 2
                wait_gather(s)
                fire_store(i, s)
                wait_store(1 - s)
                fire_gather(i + 1, 1 - s)
            return carry

        lax.fori_loop(0, (n_chunks - 2) // 2, body, 0)

        i_last = n_chunks - 1
        s_last = i_last % 2
        wait_gather(s_last)
        fire_store(i_last, s_last)
        wait_store(1 - s_last)
        wait_store(s_last)

    out = gather_k(primitives, flat_idx)
    return out.reshape(B, L, D)


# speedup vs baseline: 6.1275x; 1.1623x over previous
import functools

import jax
import jax.numpy as jnp
from jax import lax
from jax.experimental import pallas as pl
from jax.experimental.pallas import tpu as pltpu
from jax.experimental.pallas import tpu_sc as plsc


def kernel(indices, primitives):
    B, L = indices.shape
    V, D = primitives.shape
    N = B * L
    flat_idx = indices.reshape(N)
    table_flat = primitives.T.reshape(V * D)

    info = plsc.get_sparse_core_info()
    NC, NS = info.num_cores, info.num_subcores
    NW = NC * NS
    NDG = 8
    NBR = NW // NDG
    b_per_w = B // NBR
    BBLK = 8
    n_blk = b_per_w // BBLK
    DR = D // NDG

    mesh = plsc.VectorSubcoreMesh(core_axis_name="c", subcore_axis_name="s")

    @functools.partial(
        pl.kernel,
        mesh=mesh,
        compiler_params=pltpu.CompilerParams(
            use_tc_tiling_on_sc=True, needs_layout_passes=False),
        out_type=jax.ShapeDtypeStruct((B, D, L), jnp.float32),
        scratch_types=[
            pltpu.VMEM((DR * V,), jnp.float32),
            pltpu.VMEM((BBLK * L,), jnp.int32),
            pltpu.VMEM((2, DR, L), jnp.float32),
            pltpu.SemaphoreType.DMA((2,)),
        ],
    )
    def gather_k(table_hbm, idx_hbm, out_hbm, table_v, idx_v, out_v, osem):
        wid = lax.axis_index("s") * NC + lax.axis_index("c")
        jd = wid % NDG
        b0 = (wid // NDG) * b_per_w

        pltpu.sync_copy(table_hbm.at[pl.ds(jd * DR * V, DR * V)], table_v)

        def wait_out(sb):
            pltpu.make_async_copy(
                out_v.at[sb], out_hbm.at[0, pl.ds(0, DR)], osem.at[sb]).wait()

        def blk_body(g, carry):
            pltpu.sync_copy(
                idx_hbm.at[pl.ds((b0 + g * BBLK) * L, BBLK * L)], idx_v)
            for rb in range(BBLK):
                sb = rb % 2
                if rb >= 2:
                    wait_out(sb)
                else:
                    @pl.when(g > 0)
                    def _():
                        wait_out(sb)

                def j_body(j, carry):
                    idx16 = idx_v[pl.ds(rb * L + j * 16, 16)]
                    for dr in range(DR):
                        vals = plsc.load_gather(
                            table_v.at[pl.ds(dr * V, V)], [idx16])
                        out_v[sb, dr, pl.ds(j * 16, 16)] = vals
                    return carry

                lax.fori_loop(0, L // 16, j_body, 0)
                pltpu.async_copy(
                    out_v.at[sb],
                    out_hbm.at[b0 + g * BBLK + rb, pl.ds(jd * DR, DR)],
                    osem.at[sb])
            return carry

        lax.fori_loop(0, n_blk, blk_body, 0)
        wait_out(0)
        wait_out(1)

    out = gather_k(table_flat, flat_idx)
    return jnp.swapaxes(out, 1, 2)


# speedup vs baseline: 21.4909x; 3.5073x over previous
import functools

import jax
import jax.numpy as jnp
from jax import lax
from jax.experimental import pallas as pl
from jax.experimental.pallas import tpu as pltpu
from jax.experimental.pallas import tpu_sc as plsc


def kernel(indices, primitives):
    B, L = indices.shape
    V, D = primitives.shape
    N = B * L
    flat_idx = indices.reshape(N)
    table_flat = primitives.T.reshape(V * D)

    info = plsc.get_sparse_core_info()
    NC, NS = info.num_cores, info.num_subcores
    NW = NC * NS
    NDG = 8
    NBR = NW // NDG
    b_per_w = B // NBR
    BBLK = 8
    n_blk = b_per_w // BBLK
    DR = D // NDG

    mesh = plsc.VectorSubcoreMesh(core_axis_name="c", subcore_axis_name="s")

    @functools.partial(
        pl.kernel,
        mesh=mesh,
        compiler_params=pltpu.CompilerParams(
            use_tc_tiling_on_sc=True, needs_layout_passes=False),
        out_type=jax.ShapeDtypeStruct((B, D, L), jnp.float32),
        scratch_types=[
            pltpu.VMEM((DR * V,), jnp.float32),
            pltpu.VMEM((BBLK * L,), jnp.int32),
            pltpu.VMEM((2, DR, L), jnp.float32),
            pltpu.SemaphoreType.DMA((2,)),
        ],
    )
    def gather_k(table_hbm, idx_hbm, out_hbm, table_v, idx_v, out_v, osem):
        wid = lax.axis_index("s") * NC + lax.axis_index("c")
        jd = wid % NDG
        b0 = (wid // NDG) * b_per_w

        pltpu.sync_copy(table_hbm.at[pl.ds(jd * DR * V, DR * V)], table_v)

        def wait_out(sb):
            pltpu.make_async_copy(
                out_v.at[sb], out_hbm.at[0, pl.ds(0, DR)], osem.at[sb]).wait()

        def blk_body(g, carry):
            pltpu.sync_copy(
                idx_hbm.at[pl.ds((b0 + g * BBLK) * L, BBLK * L)], idx_v)
            for rb in range(BBLK):
                sb = rb % 2
                if rb >= 2:
                    wait_out(sb)
                else:
                    @pl.when(g > 0)
                    def _():
                        wait_out(sb)

                @plsc.parallel_loop(0, L // 16, unroll=4)
                def j_body(j):
                    idx16 = idx_v[pl.ds(rb * L + j * 16, 16)]
                    for dr in range(DR):
                        vals = plsc.load_gather(
                            table_v.at[pl.ds(dr * V, V)], [idx16])
                        out_v[sb, dr, pl.ds(j * 16, 16)] = vals
                pltpu.async_copy(
                    out_v.at[sb],
                    out_hbm.at[b0 + g * BBLK + rb, pl.ds(jd * DR, DR)],
                    osem.at[sb])
            return carry

        lax.fori_loop(0, n_blk, blk_body, 0)
        wait_out(0)
        wait_out(1)

    out = gather_k(table_flat, flat_idx)
    return jnp.swapaxes(out, 1, 2)
